# baseline (speedup 1.0000x reference)
import jax
import jax.numpy as jnp
from jax import lax
from jax.experimental import pallas as pl
from jax.experimental.pallas import tpu as pltpu
from jax.experimental.pallas import tpu_sc as plsc

_N = 10000
_D = 128
_NP = 10240
_NC = 2
_NS = 16
_NW = _NC * _NS
_CHUNK = 128
_R0 = 80
_R1 = 80
_PASS = 56
_NPASS = -(-_R0 // _PASS)
_ROWS = _NS * (_R0 + _R1)
_EROWS = _NS * _R0 + _NS * _R1 + _NPASS * _PASS
_EPAD = _EROWS * 128
_ROWS_PER_TILE = _NP // _NS


_DEGR = _ROWS // _NW


def _sc_deg_body(dst_hbm, degp_hbm, dstv, degv):
    c = lax.axis_index("c")
    s = lax.axis_index("s")
    wid = c * _NS + s

    zv = jnp.zeros((16,), jnp.float32)

    def _zdeg(i, carry):
        for cc in range(8):
            degv[i, pl.ds(cc * 16, 16)] = zv
        return carry
    lax.fori_loop(0, _NP // 128, _zdeg, 0)

    pltpu.sync_copy(dst_hbm.at[pl.ds(wid * _DEGR, _DEGR)], dstv)

    ones16 = jnp.ones((16,), jnp.float32)

    def _body(r, carry):
        for k in range(8):
            idx = dstv[r, pl.ds(k * 16, 16)]
            plsc.addupdate_scatter(
                degv,
                [lax.shift_right_logical(idx, 7),
                 lax.bitwise_and(idx, 127)],
                ones16)
        return carry
    lax.fori_loop(0, _DEGR, _body, 0)

    pltpu.sync_copy(degv, degp_hbm.at[wid])


def _sc_deg(dst):
    mesh = plsc.VectorSubcoreMesh(core_axis_name="c", subcore_axis_name="s")
    f = pl.kernel(
        _sc_deg_body,
        out_type=jax.ShapeDtypeStruct((_NW, _NP // 128, 128), jnp.float32),
        mesh=mesh,
        compiler_params=pltpu.CompilerParams(needs_layout_passes=False),
        scratch_types=[
            pltpu.VMEM((_DEGR, 128), jnp.int32),
            pltpu.VMEM((_NP // 128, 128), jnp.float32),
        ],
    )
    return f(dst)


def _sc_segsum_body(src_hbm, dst_hbm, y_hbm, degp_hbm, agg_hbm,
                    srcv, dstv, rows, acc, gsem, ssem):
    del degp_hbm
    c = lax.axis_index("c")
    s = lax.axis_index("s")

    base = jnp.where(c == 0, s * _R0, _NS * _R0 + s * _R1)
    rc = jnp.where(c == 0, _R0, _R1)

    zv = jnp.zeros((16,), jnp.float32)

    def _zrow(r, carry):
        for cc in range(8):
            rows[0, r, pl.ds(cc * 16, 16)] = zv
        return carry
    lax.fori_loop(0, _CHUNK, _zrow, 0)
    for k in range(_ROWS_PER_TILE // _CHUNK):
        pltpu.sync_copy(rows.at[0],
                        acc.at[pl.ds(s * _ROWS_PER_TILE + k * _CHUNK, _CHUNK)])

    plsc.subcore_barrier()

    def _start(r, b):
        pltpu.async_copy(y_hbm.at[srcv.at[r]], rows.at[b], gsem)

    def _wait(b):
        pltpu.make_async_copy(y_hbm.at[srcv.at[0]], rows.at[b], gsem).wait()

    def _scatter(r, b):
        pltpu.async_copy(rows.at[b], acc.at[dstv.at[r]], ssem, add=True)

    def _scatter_wait():
        pltpu.make_async_copy(rows.at[0], acc.at[dstv.at[0]], ssem).wait()

    for p in range(_NPASS):
        rcp = jnp.clip(rc - p * _PASS, 0, _PASS)

        @pl.when(rcp > 0)
        def _():
            pltpu.sync_copy(src_hbm.at[pl.ds(base + p * _PASS, _PASS)], srcv)
            pltpu.sync_copy(dst_hbm.at[pl.ds(base + p * _PASS, _PASS)], dstv)
            _start(0, 0)

            def _body(g, carry):
                for b in range(2):
                    r = 2 * g + b
                    _wait(b)
                    _scatter(r, b)

                    @pl.when(r >= 1)
                    def _():
                        _scatter_wait()

                    @pl.when(r + 1 < rcp)
                    def _():
                        _start(r + 1, 1 - b)
                return carry
            lax.fori_loop(0, (rcp + 1) // 2, _body, 0)
            _scatter_wait()

    plsc.subcore_barrier()
    pltpu.sync_copy(acc.at[pl.ds(s * _ROWS_PER_TILE, _ROWS_PER_TILE)],
                    agg_hbm.at[c, pl.ds(s * _ROWS_PER_TILE, _ROWS_PER_TILE)])


def _sc_segsum(src, dst, y, degp):
    mesh = plsc.VectorSubcoreMesh(core_axis_name="c", subcore_axis_name="s")
    f = pl.kernel(
        _sc_segsum_body,
        out_type=jax.ShapeDtypeStruct((_NC, _NP, _D), jnp.float32),
        mesh=mesh,
        compiler_params=pltpu.CompilerParams(needs_layout_passes=False),
        scratch_types=[
            pltpu.VMEM((_PASS, 128), jnp.int32),
            pltpu.VMEM((_PASS, 128), jnp.int32),
            pltpu.VMEM((2, _CHUNK, _D), jnp.float32),
            pltpu.VMEM_SHARED((_NP, _D), jnp.float32),
            pltpu.SemaphoreType.DMA,
            pltpu.SemaphoreType.DMA,
        ],
    )
    return f(src, dst, y, degp)


_BLK = 1024
_GRID = _NP // _BLK


def _mm_body(x_ref, wl_ref, wr_ref, b_ref, y_ref, z_ref):
    xb = x_ref[...]
    y_ref[...] = jnp.dot(xb, wl_ref[...], preferred_element_type=jnp.float32)
    z_ref[...] = (jnp.dot(xb, wr_ref[...], preferred_element_type=jnp.float32)
                  + b_ref[...])


def _mm(x, wl_t, wr_t, b):
    return pl.pallas_call(
        _mm_body,
        grid=(_GRID,),
        in_specs=[
            pl.BlockSpec((_BLK, _D), lambda i: (i, 0)),
            pl.BlockSpec((_D, _D), lambda i: (0, 0)),
            pl.BlockSpec((_D, _D), lambda i: (0, 0)),
            pl.BlockSpec((_D,), lambda i: (0,)),
        ],
        out_specs=[
            pl.BlockSpec((_BLK, _D), lambda i: (i, 0)),
            pl.BlockSpec((_BLK, _D), lambda i: (i, 0)),
        ],
        out_shape=[
            jax.ShapeDtypeStruct((_NP, _D), jnp.float32),
            jax.ShapeDtypeStruct((_NP, _D), jnp.float32),
        ],
    )(x, wl_t, wr_t, b)


def _combine_mm_body(agg_ref, degp_ref, z_ref, wl_ref, wr_ref, b_ref,
                     y_ref, z2_ref, invd_ref):
    deg = jnp.sum(degp_ref[...], axis=0)
    invd = 1.0 / jnp.maximum(deg, 1.0)
    h = jnp.maximum((agg_ref[0] + agg_ref[1]) * invd[:, None] + z_ref[...],
                    0.0)
    y_ref[...] = jnp.dot(h, wl_ref[...], preferred_element_type=jnp.float32)
    z2_ref[...] = (jnp.dot(h, wr_ref[...], preferred_element_type=jnp.float32)
                   + b_ref[...])
    invd_ref[...] = invd


def _combine_mm(agg, degp, z, wl_t, wr_t, b):
    return pl.pallas_call(
        _combine_mm_body,
        grid=(_GRID,),
        in_specs=[
            pl.BlockSpec((_NC, _BLK, _D), lambda i: (0, i, 0)),
            pl.BlockSpec((_NW, _BLK), lambda i: (0, i)),
            pl.BlockSpec((_BLK, _D), lambda i: (i, 0)),
            pl.BlockSpec((_D, _D), lambda i: (0, 0)),
            pl.BlockSpec((_D, _D), lambda i: (0, 0)),
            pl.BlockSpec((_D,), lambda i: (0,)),
        ],
        out_specs=[
            pl.BlockSpec((_BLK, _D), lambda i: (i, 0)),
            pl.BlockSpec((_BLK, _D), lambda i: (i, 0)),
            pl.BlockSpec((_BLK,), lambda i: (i,)),
        ],
        out_shape=[
            jax.ShapeDtypeStruct((_NP, _D), jnp.float32),
            jax.ShapeDtypeStruct((_NP, _D), jnp.float32),
            jax.ShapeDtypeStruct((_NP,), jnp.float32),
        ],
    )(agg, degp, z, wl_t, wr_t, b)


def _final_body(agg_ref, invd_ref, z_ref, out_ref):
    out_ref[...] = jnp.maximum(
        (agg_ref[0] + agg_ref[1]) * invd_ref[...][:, None] + z_ref[...], 0.0)


def _final(agg, invd, z):
    return pl.pallas_call(
        _final_body,
        grid=(_GRID,),
        in_specs=[
            pl.BlockSpec((_NC, _BLK, _D), lambda i: (0, i, 0)),
            pl.BlockSpec((_BLK,), lambda i: (i,)),
            pl.BlockSpec((_BLK, _D), lambda i: (i, 0)),
        ],
        out_specs=pl.BlockSpec((_BLK, _D), lambda i: (i, 0)),
        out_shape=jax.ShapeDtypeStruct((_NP, _D), jnp.float32),
    )(agg, invd, z)


def kernel(x, edge_index, W1l, b1l, W1r, W2l, b2l, W2r):
    e = edge_index.shape[1]
    pad = _EPAD - e
    xp = jnp.pad(x, ((0, _NP - _N), (0, 0)))
    pad_src = (jnp.arange(pad, dtype=jnp.int32) * 37) % _N
    pad_dst = _N + jnp.arange(pad, dtype=jnp.int32) % (_NP - _N)
    src = jnp.concatenate([edge_index[0], pad_src])
    dst = jnp.concatenate([edge_index[1], pad_dst])
    src = src.reshape(_EROWS, 128)
    dst = dst.reshape(_EROWS, 128)

    degp3 = _sc_deg(dst)
    degp = degp3.reshape(_NW, _NP)

    y1, z1 = _mm(xp, W1l.T, W1r.T, b1l)
    agg1 = _sc_segsum(src, dst, y1, degp3)
    y2, z2, invd = _combine_mm(agg1, degp, z1, W2l.T, W2r.T, b2l)

    agg2 = _sc_segsum(src, dst, y2, degp3)
    out = _final(agg2, invd, z2)
    return out[:_N]


# speedup vs baseline: 1.0076x; 1.0076x over previous
import jax
import jax.numpy as jnp
from jax import lax
from jax.experimental import pallas as pl
from jax.experimental.pallas import tpu as pltpu
from jax.experimental.pallas import tpu_sc as plsc

_N = 10000
_D = 128
_NP = 10240
_NC = 2
_NS = 16
_NW = _NC * _NS
_CHUNK = 128
_R0 = 80
_R1 = 80
_PASS = 56
_NPASS = -(-_R0 // _PASS)
_ROWS = _NS * (_R0 + _R1)
_EROWS = _NS * _R0 + _NS * _R1 + _NPASS * _PASS
_EPAD = _EROWS * 128
_ROWS_PER_TILE = _NP // _NS


_DEGR = _ROWS // _NW


def _sc_deg_body(dst_hbm, degp_hbm, dstv, degv):
    c = lax.axis_index("c")
    s = lax.axis_index("s")
    wid = c * _NS + s

    zv = jnp.zeros((16,), jnp.float32)

    def _zdeg(i, carry):
        for cc in range(8):
            degv[i, pl.ds(cc * 16, 16)] = zv
        return carry
    lax.fori_loop(0, _NP // 128, _zdeg, 0)

    pltpu.sync_copy(dst_hbm.at[pl.ds(wid * _DEGR, _DEGR)], dstv)

    ones16 = jnp.ones((16,), jnp.float32)

    def _body(r, carry):
        for k in range(8):
            idx = dstv[r, pl.ds(k * 16, 16)]
            plsc.addupdate_scatter(
                degv,
                [lax.shift_right_logical(idx, 7),
                 lax.bitwise_and(idx, 127)],
                ones16)
        return carry
    lax.fori_loop(0, _DEGR, _body, 0)

    pltpu.sync_copy(degv, degp_hbm.at[wid])


def _sc_deg(dst):
    mesh = plsc.VectorSubcoreMesh(core_axis_name="c", subcore_axis_name="s")
    f = pl.kernel(
        _sc_deg_body,
        out_type=jax.ShapeDtypeStruct((_NW, _NP // 128, 128), jnp.float32),
        mesh=mesh,
        compiler_params=pltpu.CompilerParams(needs_layout_passes=False),
        scratch_types=[
            pltpu.VMEM((_DEGR, 128), jnp.int32),
            pltpu.VMEM((_NP // 128, 128), jnp.float32),
        ],
    )
    return f(dst)


def _sc_segsum_body(src_hbm, dst_hbm, y_hbm, degp_hbm, agg_hbm,
                    srcv, dstv, rows, acc, gsem):
    del degp_hbm
    c = lax.axis_index("c")
    s = lax.axis_index("s")

    base = jnp.where(c == 0, s * _R0, _NS * _R0 + s * _R1)
    rc = jnp.where(c == 0, _R0, _R1)

    zv = jnp.zeros((16,), jnp.float32)

    def _zrow(r, carry):
        for cc in range(8):
            rows[0, r, pl.ds(cc * 16, 16)] = zv
        return carry
    lax.fori_loop(0, _CHUNK, _zrow, 0)
    for k in range(_ROWS_PER_TILE // _CHUNK):
        pltpu.sync_copy(rows.at[0],
                        acc.at[pl.ds(s * _ROWS_PER_TILE + k * _CHUNK, _CHUNK)])

    plsc.subcore_barrier()

    def _start(r, b):
        pltpu.async_copy(y_hbm.at[srcv.at[r]], rows.at[b], gsem)

    def _wait(b):
        pltpu.make_async_copy(y_hbm.at[srcv.at[0]], rows.at[b], gsem).wait()

    for p in range(_NPASS):
        rcp = jnp.clip(rc - p * _PASS, 0, _PASS)

        @pl.when(rcp > 0)
        def _():
            pltpu.sync_copy(src_hbm.at[pl.ds(base + p * _PASS, _PASS)], srcv)
            pltpu.sync_copy(dst_hbm.at[pl.ds(base + p * _PASS, _PASS)], dstv)
            _start(0, 0)

            def _body(g, carry):
                for b in range(2):
                    r = 2 * g + b
                    _wait(b)

                    @pl.when(r + 1 < rcp)
                    def _():
                        _start(r + 1, 1 - b)

                    pltpu.sync_copy(rows.at[b], acc.at[dstv.at[r]],
                                    add=True)
                return carry
            lax.fori_loop(0, (rcp + 1) // 2, _body, 0)

    plsc.subcore_barrier()
    pltpu.sync_copy(acc.at[pl.ds(s * _ROWS_PER_TILE, _ROWS_PER_TILE)],
                    agg_hbm.at[c, pl.ds(s * _ROWS_PER_TILE, _ROWS_PER_TILE)])


def _sc_segsum(src, dst, y, degp):
    mesh = plsc.VectorSubcoreMesh(core_axis_name="c", subcore_axis_name="s")
    f = pl.kernel(
        _sc_segsum_body,
        out_type=jax.ShapeDtypeStruct((_NC, _NP, _D), jnp.float32),
        mesh=mesh,
        compiler_params=pltpu.CompilerParams(needs_layout_passes=False),
        scratch_types=[
            pltpu.VMEM((_PASS, 128), jnp.int32),
            pltpu.VMEM((_PASS, 128), jnp.int32),
            pltpu.VMEM((2, _CHUNK, _D), jnp.float32),
            pltpu.VMEM_SHARED((_NP, _D), jnp.float32),
            pltpu.SemaphoreType.DMA,
        ],
    )
    return f(src, dst, y, degp)


_BLK = 1024
_GRID = _NP // _BLK
_MMBLK = 1000
_MMGRID = _N // _MMBLK


def _mm_body(x_ref, wl_ref, wr_ref, b_ref, y_ref, z_ref):
    xb = x_ref[...]
    y_ref[...] = jnp.dot(xb, wl_ref[...], preferred_element_type=jnp.float32)
    z_ref[...] = (jnp.dot(xb, wr_ref[...], preferred_element_type=jnp.float32)
                  + b_ref[...])


def _mm(x, wl_t, wr_t, b):
    return pl.pallas_call(
        _mm_body,
        grid=(_MMGRID,),
        in_specs=[
            pl.BlockSpec((_MMBLK, _D), lambda i: (i, 0)),
            pl.BlockSpec((_D, _D), lambda i: (0, 0)),
            pl.BlockSpec((_D, _D), lambda i: (0, 0)),
            pl.BlockSpec((_D,), lambda i: (0,)),
        ],
        out_specs=[
            pl.BlockSpec((_MMBLK, _D), lambda i: (i, 0)),
            pl.BlockSpec((_MMBLK, _D), lambda i: (i, 0)),
        ],
        out_shape=[
            jax.ShapeDtypeStruct((_NP, _D), jnp.float32),
            jax.ShapeDtypeStruct((_NP, _D), jnp.float32),
        ],
    )(x, wl_t, wr_t, b)


def _combine_mm_body(agg_ref, degp_ref, z_ref, wl_ref, wr_ref, b_ref,
                     y_ref, z2_ref, invd_ref):
    deg = jnp.sum(degp_ref[...], axis=0)
    invd = 1.0 / jnp.maximum(deg, 1.0)
    h = jnp.maximum((agg_ref[0] + agg_ref[1]) * invd[:, None] + z_ref[...],
                    0.0)
    y_ref[...] = jnp.dot(h, wl_ref[...], preferred_element_type=jnp.float32)
    z2_ref[...] = (jnp.dot(h, wr_ref[...], preferred_element_type=jnp.float32)
                   + b_ref[...])
    invd_ref[...] = invd


def _combine_mm(agg, degp, z, wl_t, wr_t, b):
    return pl.pallas_call(
        _combine_mm_body,
        grid=(_GRID,),
        in_specs=[
            pl.BlockSpec((_NC, _BLK, _D), lambda i: (0, i, 0)),
            pl.BlockSpec((_NW, _BLK), lambda i: (0, i)),
            pl.BlockSpec((_BLK, _D), lambda i: (i, 0)),
            pl.BlockSpec((_D, _D), lambda i: (0, 0)),
            pl.BlockSpec((_D, _D), lambda i: (0, 0)),
            pl.BlockSpec((_D,), lambda i: (0,)),
        ],
        out_specs=[
            pl.BlockSpec((_BLK, _D), lambda i: (i, 0)),
            pl.BlockSpec((_BLK, _D), lambda i: (i, 0)),
            pl.BlockSpec((_BLK,), lambda i: (i,)),
        ],
        out_shape=[
            jax.ShapeDtypeStruct((_NP, _D), jnp.float32),
            jax.ShapeDtypeStruct((_NP, _D), jnp.float32),
            jax.ShapeDtypeStruct((_NP,), jnp.float32),
        ],
    )(agg, degp, z, wl_t, wr_t, b)


def _final_body(agg_ref, invd_ref, z_ref, out_ref):
    out_ref[...] = jnp.maximum(
        (agg_ref[0] + agg_ref[1]) * invd_ref[...][:, None] + z_ref[...], 0.0)


def _final(agg, invd, z):
    return pl.pallas_call(
        _final_body,
        grid=(_GRID,),
        in_specs=[
            pl.BlockSpec((_NC, _BLK, _D), lambda i: (0, i, 0)),
            pl.BlockSpec((_BLK,), lambda i: (i,)),
            pl.BlockSpec((_BLK, _D), lambda i: (i, 0)),
        ],
        out_specs=pl.BlockSpec((_BLK, _D), lambda i: (i, 0)),
        out_shape=jax.ShapeDtypeStruct((_NP, _D), jnp.float32),
    )(agg, invd, z)


def kernel(x, edge_index, W1l, b1l, W1r, W2l, b2l, W2r):
    e = edge_index.shape[1]
    pad = _EPAD - e
    pad_src = (jnp.arange(pad, dtype=jnp.int32) * 37) % _N
    pad_dst = _N + jnp.arange(pad, dtype=jnp.int32) % (_NP - _N)
    src = jnp.concatenate([edge_index[0], pad_src])
    dst = jnp.concatenate([edge_index[1], pad_dst])
    src = src.reshape(_EROWS, 128)
    dst = dst.reshape(_EROWS, 128)

    degp3 = _sc_deg(dst)
    degp = degp3.reshape(_NW, _NP)

    y1, z1 = _mm(x, W1l.T, W1r.T, b1l)
    agg1 = _sc_segsum(src, dst, y1, degp3)
    y2, z2, invd = _combine_mm(agg1, degp, z1, W2l.T, W2r.T, b2l)

    agg2 = _sc_segsum(src, dst, y2, degp3)
    out = _final(agg2, invd, z2)
    return out[:_N]
